# baseline (speedup 1.0000x reference)
import functools

import jax
import jax.numpy as jnp
from jax import lax
from jax.experimental import pallas as pl
from jax.experimental.pallas import tpu as pltpu
from jax.experimental.pallas import tpu_sc as plsc

_NC, _NS = 2, 16
_NW = _NC * _NS
_D = 128
_EPS = 1e-5


def _ln(h, g, b):
    mu = h.mean(-1, keepdims=True)
    var = ((h - mu) ** 2).mean(-1, keepdims=True)
    return (h - mu) / jnp.sqrt(var + _EPS) * g + b


def _silu(x):
    return x * jax.nn.sigmoid(x)


def _dot(a, b):
    return jnp.dot(a, b, preferred_element_type=jnp.float32)


def _full2(shape):
    return pl.BlockSpec(shape, lambda i: (0, 0))


def _rows(blk, d):
    return pl.BlockSpec((blk, d), lambda i: (i, 0))


def _mlp4_body(x_ref, w1, b1, g, bl, w2, b2, o_ref):
    h = _dot(x_ref[...], w1[...]) + b1[...]
    h = _silu(_ln(h, g[...], bl[...]))
    o_ref[...] = _dot(h, w2[...]) + b2[...]


def _tc_mlp4(x, p1, pln, p2, blk):
    n, din = x.shape
    dout = p2["W"].shape[1]
    dmid = p1["W"].shape[1]
    return pl.pallas_call(
        _mlp4_body,
        grid=(n // blk,),
        in_specs=[_rows(blk, din), _full2((din, dmid)), _full2((1, dmid)),
                  _full2((1, dmid)), _full2((1, dmid)), _full2((dmid, dout)),
                  _full2((1, dout))],
        out_specs=_rows(blk, dout),
        out_shape=jax.ShapeDtypeStruct((n, dout), jnp.float32),
    )(x, p1["W"], p1["b"][None], pln["g"][None], pln["b"][None],
      p2["W"], p2["b"][None])


def _pool_mlp4_body(pairs_ref, w_ref, aux_ref, w1a, w1b, b1, g, bl, w2, b2,
                    o_ref):
    pr = pairs_ref[...]
    wt = w_ref[...]
    k_ = wt.shape[1]
    pooled = pr[:, 0:_D] * wt[:, 0:1]
    for k in range(1, k_):
        pooled = pooled + pr[:, k * _D:(k + 1) * _D] * wt[:, k:k + 1]
    h = _dot(pooled, w1a[...]) + b1[...]
    if aux_ref is not None:
        h = h + _dot(aux_ref[...], w1b[...])
    h = _silu(_ln(h, g[...], bl[...]))
    o_ref[...] = _dot(h, w2[...]) + b2[...]


def _tc_pool_mlp4(pairs, weights, aux, w1a, w1b, b1, pln, p2, blk):
    n, kd = pairs.shape
    k_ = weights.shape[1]
    dout = p2["W"].shape[1]
    if aux is None:
        def body(pairs_ref, w_ref, w1a_, b1_, g_, bl_, w2_, b2_, o_ref):
            _pool_mlp4_body(pairs_ref, w_ref, None, w1a_, None, b1_, g_, bl_,
                            w2_, b2_, o_ref)
        in_specs = [_rows(blk, kd), _rows(blk, k_), _full2((_D, _D)),
                    _full2((1, _D)), _full2((1, _D)), _full2((1, _D)),
                    _full2((_D, dout)), _full2((1, dout))]
        args = (pairs, weights, w1a, b1[None], pln["g"][None], pln["b"][None],
                p2["W"], p2["b"][None])
    else:
        body = _pool_mlp4_body
        da = aux.shape[1]
        in_specs = [_rows(blk, kd), _rows(blk, k_), _rows(blk, da),
                    _full2((_D, _D)), _full2((da, _D)), _full2((1, _D)),
                    _full2((1, _D)), _full2((1, _D)), _full2((_D, dout)),
                    _full2((1, dout))]
        args = (pairs, weights, aux, w1a, w1b, b1[None], pln["g"][None],
                pln["b"][None], p2["W"], p2["b"][None])
    return pl.pallas_call(
        body,
        grid=(n // blk,),
        in_specs=in_specs,
        out_specs=_rows(blk, dout),
        out_shape=jax.ShapeDtypeStruct((n, dout), jnp.float32),
    )(*args)


def _xdxs_body(x_ref, wd, ws, be, xd_ref, xs_ref):
    x = x_ref[...]
    xd_ref[...] = _dot(x, wd[...]) + be[...]
    xs_ref[...] = _dot(x, ws[...])


def _tc_xdxs(x, wd, ws, be, blk):
    n = x.shape[0]
    return pl.pallas_call(
        _xdxs_body,
        grid=(n // blk,),
        in_specs=[_rows(blk, _D), _full2((_D, _D)), _full2((_D, _D)),
                  _full2((1, _D))],
        out_specs=[_rows(blk, _D), _rows(blk, _D)],
        out_shape=[jax.ShapeDtypeStruct((n, _D), jnp.float32),
                   jax.ShapeDtypeStruct((n, _D), jnp.float32)],
    )(x, wd, ws, be[None])


def _msg_body(pre_ref, ea_ref, wea, g, bl, o_ref):
    h = pre_ref[...] + _dot(ea_ref[...], wea[...])
    o_ref[...] = _silu(_ln(h, g[...], bl[...]))


def _tc_msg(pre, ea_p, wea_p, pln, blk):
    n = pre.shape[0]
    da = ea_p.shape[1]
    return pl.pallas_call(
        _msg_body,
        grid=(n // blk,),
        in_specs=[_rows(blk, _D), _rows(blk, da), _full2((da, _D)),
                  _full2((1, _D)), _full2((1, _D))],
        out_specs=_rows(blk, _D),
        out_shape=jax.ShapeDtypeStruct((n, _D), jnp.float32),
    )(pre, ea_p, wea_p, pln["g"][None], pln["b"][None])


def _node_body(x_ref, a0_ref, a1_ref, w1a, w1b, b1, g, bl, w2, b2, o_ref):
    x = x_ref[...]
    agg = a0_ref[...] + a1_ref[...]
    h = _dot(x, w1a[...]) + _dot(agg, w1b[...]) + b1[...]
    h = _silu(_ln(h, g[...], bl[...]))
    o_ref[...] = x + _dot(h, w2[...]) + b2[...]


def _tc_node(x, a0, a1, w1a, w1b, b1, pln, p2, blk):
    n = x.shape[0]
    return pl.pallas_call(
        _node_body,
        grid=(n // blk,),
        in_specs=[_rows(blk, _D)] * 3 + [_full2((_D, _D)), _full2((_D, _D)),
                  _full2((1, _D)), _full2((1, _D)), _full2((1, _D)),
                  _full2((_D, _D)), _full2((1, _D))],
        out_specs=_rows(blk, _D),
        out_shape=jax.ShapeDtypeStruct((n, _D), jnp.float32),
    )(x, a0, a1, w1a, w1b, b1[None], pln["g"][None], pln["b"][None],
      p2["W"], p2["b"][None])


def _segsum_body(dst_ref, msg_ref, o_ref):
    n_i = pl.program_id(0)
    e_i = pl.program_id(1)
    nblk = o_ref.shape[0]
    eblk = msg_ref.shape[0]

    @pl.when(e_i == 0)
    def _():
        o_ref[...] = jnp.zeros_like(o_ref)

    rows = jax.lax.broadcasted_iota(jnp.int32, (nblk, eblk), 0) + n_i * nblk
    onehot_t = (rows == dst_ref[...]).astype(jnp.float32)
    o_ref[...] += _dot(onehot_t, msg_ref[...])


def _tc_segsum(msg, dst, n_seg, nblk, eblk):
    e = msg.shape[0]
    n_seg_p = -(-n_seg // nblk) * nblk
    return pl.pallas_call(
        _segsum_body,
        grid=(n_seg_p // nblk, e // eblk),
        in_specs=[pl.BlockSpec((1, eblk), lambda n, i: (0, i)),
                  pl.BlockSpec((eblk, _D), lambda n, i: (i, 0))],
        out_specs=pl.BlockSpec((nblk, _D), lambda n, i: (n, 0)),
        out_shape=jax.ShapeDtypeStruct((n_seg_p, _D), jnp.float32),
    )(dst[None], msg)[:n_seg]


_SC_MESH = plsc.VectorSubcoreMesh(core_axis_name="c", subcore_axis_name="s")


def _sc_gather(table, idx_flat, ch):
    b = idx_flat.shape[0]
    d = table.shape[1]
    b_per_w = b // _NW
    n_ch = b_per_w // ch

    @functools.partial(
        pl.kernel, mesh=_SC_MESH,
        out_type=jax.ShapeDtypeStruct((b, d), jnp.float32),
        scratch_types=[pltpu.VMEM((ch,), jnp.int32),
                       pltpu.VMEM((ch, d), jnp.float32),
                       pltpu.SemaphoreType.DMA],
    )
    def k(table_hbm, idx_hbm, out_hbm, idx_v, rows_v, sem):
        wid = lax.axis_index("s") * _NC + lax.axis_index("c")
        base = wid * b_per_w

        def body(c, carry):
            off = base + c * ch
            pltpu.sync_copy(idx_hbm.at[pl.ds(off, ch)], idx_v)
            pltpu.async_copy(table_hbm.at[idx_v], rows_v, sem).wait()
            pltpu.sync_copy(rows_v, out_hbm.at[pl.ds(off, ch)])
            return carry

        lax.fori_loop(0, n_ch, body, 0)

    return k(table, idx_flat)


def _sc_gather2_add(xd, xs, dst, src, ch):
    b = dst.shape[0]
    b_per_w = b // _NW
    n_ch = b_per_w // ch

    @functools.partial(
        pl.kernel, mesh=_SC_MESH,
        out_type=jax.ShapeDtypeStruct((b, _D), jnp.float32),
        scratch_types=[pltpu.VMEM((ch,), jnp.int32),
                       pltpu.VMEM((ch, _D), jnp.float32),
                       pltpu.VMEM((ch, _D), jnp.float32),
                       pltpu.SemaphoreType.DMA],
    )
    def k(xd_hbm, xs_hbm, dst_hbm, src_hbm, out_hbm, idx_v, rd, rs, sem):
        wid = lax.axis_index("s") * _NC + lax.axis_index("c")
        base = wid * b_per_w

        def body(c, carry):
            off = base + c * ch
            pltpu.sync_copy(dst_hbm.at[pl.ds(off, ch)], idx_v)
            pltpu.async_copy(xd_hbm.at[idx_v], rd, sem).wait()
            pltpu.sync_copy(src_hbm.at[pl.ds(off, ch)], idx_v)
            pltpu.async_copy(xs_hbm.at[idx_v], rs, sem).wait()

            def add_row(r, carry2):
                for f in range(_D // 16):
                    sl = pl.ds(f * 16, 16)
                    rd[r, sl] = rd[r, sl] + rs[r, sl]
                return carry2

            lax.fori_loop(0, ch, add_row, 0)
            pltpu.sync_copy(rd, out_hbm.at[pl.ds(off, ch)])
            return carry

        lax.fori_loop(0, n_ch, body, 0)

    return k(xd, xs, dst, src)


def _sc_scatter_add(msg, eidx, n_seg_p, ch):
    b = msg.shape[0]
    b_per_w = b // _NW
    n_ch = b_per_w // ch
    rows_per_tile = n_seg_p // _NS
    zcopy = ch
    n_z = rows_per_tile // zcopy
    msg1 = msg.reshape(-1)

    @functools.partial(
        pl.kernel, mesh=_SC_MESH,
        out_type=jax.ShapeDtypeStruct((_NC, n_seg_p * _D), jnp.float32),
        scratch_types=[pltpu.VMEM((ch * _D,), jnp.int32),
                       pltpu.VMEM((ch * _D,), jnp.float32),
                       pltpu.VMEM((zcopy * _D,), jnp.float32),
                       pltpu.VMEM_SHARED((n_seg_p * _D,), jnp.float32),
                       pltpu.SemaphoreType.DMA],
    )
    def k(msg_hbm, eidx_hbm, out_hbm, eidx_v, rows_v, zero_v, acc, sem):
        cid = lax.axis_index("c")
        sid = lax.axis_index("s")
        wid = sid * _NC + cid
        base = wid * b_per_w * _D

        def zf(r, carry):
            zero_v[pl.ds(r * 16, 16)] = jnp.zeros((16,), jnp.float32)
            return carry

        lax.fori_loop(0, zcopy * _D // 16, zf, 0)
        zh = [pltpu.async_copy(
                  zero_v,
                  acc.at[pl.ds((sid * rows_per_tile + j * zcopy) * _D,
                               zcopy * _D)],
                  sem) for j in range(n_z)]
        for h in zh:
            h.wait()
        plsc.subcore_barrier()
        pl.delay(50000)

        def body(c, carry):
            off = base + c * ch * _D
            pltpu.sync_copy(eidx_hbm.at[pl.ds(off, ch * _D)], eidx_v)
            pltpu.sync_copy(msg_hbm.at[pl.ds(off, ch * _D)], rows_v)
            pltpu.sync_copy(rows_v, acc.at[eidx_v], add=True)
            return carry

        lax.fori_loop(0, n_ch, body, 0)
        plsc.subcore_barrier()
        pl.delay(50000)

        dh = [pltpu.async_copy(
                  acc.at[pl.ds((sid * rows_per_tile + j * zcopy) * _D,
                               zcopy * _D)],
                  out_hbm.at[cid].at[pl.ds((sid * rows_per_tile
                                            + j * zcopy) * _D, zcopy * _D)],
                  sem) for j in range(n_z)]
        for h in dh:
            h.wait()

    return k(msg1, eidx)


def _pad_cols(a, to):
    n, c = a.shape
    return jnp.concatenate([a, jnp.zeros((n, to - c), a.dtype)], axis=1)


def kernel(grid_input, mesh_features, edge_index, edge_attr, g2m_indices,
           g2m_weights, m2g_indices, m2g_weights, params):
    bsz, ng, gd = grid_input.shape
    nm = mesh_features.shape[0]
    e = edge_index.shape[1]
    kk = g2m_indices.shape[1]
    p = params

    x_g = grid_input.reshape(ng, gd)

    gp = _tc_mlp4(x_g, p["enc_grid_l1"], p["enc_grid_ln"], p["enc_grid_l2"],
                  blk=2000)

    g2m_flat = g2m_indices.reshape(-1).astype(jnp.int32)
    npad = (-g2m_flat.shape[0]) % (8 * _NW * kk)
    g2m_flat = jnp.concatenate(
        [g2m_flat, jnp.zeros((npad,), jnp.int32)]) if npad else g2m_flat
    pairs = _sc_gather(gp, g2m_flat, ch=256)
    pairs = pairs[:nm * kk].reshape(nm, kk * _D)

    mf_p = _pad_cols(mesh_features, 8)
    w1 = p["enc_comb_l1"]["W"]
    w1a, w1b = w1[:_D], _pad_cols(w1[_D:].T, 8).T
    x = _tc_pool_mlp4(pairs, g2m_weights, mf_p, w1a, w1b,
                      p["enc_comb_l1"]["b"], p["enc_comb_ln"],
                      p["enc_comb_l2"], blk=2000)

    src = edge_index[0].astype(jnp.int32)
    dst = edge_index[1].astype(jnp.int32)
    ea_p = _pad_cols(edge_attr, 8)
    nm_p = -(-nm // (8 * _NS)) * (8 * _NS)
    eidx = (dst[:, None] * _D + jnp.arange(_D, dtype=jnp.int32)[None]).reshape(-1)

    for lp in p["layers"]:
        we = lp["edge_l"]["W"]
        wd, ws = we[:_D], we[_D:2 * _D]
        wea_p = _pad_cols(we[2 * _D:].T, 8).T
        xd, xs = _tc_xdxs(x, wd, ws, lp["edge_l"]["b"], blk=2000)
        pre = _sc_gather2_add(xd, xs, dst, src, ch=200)
        msg = _tc_msg(pre, ea_p, wea_p, lp["edge_ln"], blk=2000)
        agg = _tc_segsum(msg, dst, nm, nblk=512, eblk=3200)
        wn = lp["node_l1"]["W"]
        x = _tc_node(x, agg, jnp.zeros_like(agg), wn[:_D], wn[_D:],
                     lp["node_l1"]["b"], lp["node_ln"], lp["node_l2"],
                     blk=2000)

    m2g_flat = m2g_indices.reshape(-1).astype(jnp.int32)
    pairs2 = _sc_gather(x, m2g_flat, ch=200)
    pairs2 = pairs2.reshape(ng, kk * _D)
    out = _tc_pool_mlp4(pairs2, m2g_weights, None, p["dec_l1"]["W"], None,
                        p["dec_l1"]["b"], p["dec_ln"], p["dec_l2"], blk=2000)
    return out.reshape(bsz, ng, gd)


# speedup vs baseline: 1.2684x; 1.2684x over previous
import functools

import jax
import jax.numpy as jnp
from jax import lax
from jax.experimental import pallas as pl
from jax.experimental.pallas import tpu as pltpu
from jax.experimental.pallas import tpu_sc as plsc

_NC, _NS = 2, 16
_NW = _NC * _NS
_D = 128
_EPS = 1e-5


def _ln(h, g, b):
    mu = h.mean(-1, keepdims=True)
    var = ((h - mu) ** 2).mean(-1, keepdims=True)
    return (h - mu) / jnp.sqrt(var + _EPS) * g + b


def _silu(x):
    return x * jax.nn.sigmoid(x)


def _dot(a, b):
    return jnp.dot(a, b, preferred_element_type=jnp.float32)


def _full2(shape):
    return pl.BlockSpec(shape, lambda i: (0, 0))


def _rows(blk, d):
    return pl.BlockSpec((blk, d), lambda i: (i, 0))


def _mlp4_body(x_ref, w1, b1, g, bl, w2, b2, o_ref):
    h = _dot(x_ref[...], w1[...]) + b1[...]
    h = _silu(_ln(h, g[...], bl[...]))
    o_ref[...] = _dot(h, w2[...]) + b2[...]


def _tc_mlp4(x, p1, pln, p2, blk):
    n, din = x.shape
    dout = p2["W"].shape[1]
    dmid = p1["W"].shape[1]
    return pl.pallas_call(
        _mlp4_body,
        grid=(n // blk,),
        in_specs=[_rows(blk, din), _full2((din, dmid)), _full2((1, dmid)),
                  _full2((1, dmid)), _full2((1, dmid)), _full2((dmid, dout)),
                  _full2((1, dout))],
        out_specs=_rows(blk, dout),
        out_shape=jax.ShapeDtypeStruct((n, dout), jnp.float32),
    )(x, p1["W"], p1["b"][None], pln["g"][None], pln["b"][None],
      p2["W"], p2["b"][None])


def _pool_mlp4_body(pairs_ref, w_ref, aux_ref, w1a, w1b, b1, g, bl, w2, b2,
                    o_ref):
    pr = pairs_ref[...]
    wt = w_ref[...]
    k_ = wt.shape[1]
    pooled = pr[:, 0:_D] * wt[:, 0:1]
    for k in range(1, k_):
        pooled = pooled + pr[:, k * _D:(k + 1) * _D] * wt[:, k:k + 1]
    h = _dot(pooled, w1a[...]) + b1[...]
    if aux_ref is not None:
        h = h + _dot(aux_ref[...], w1b[...])
    h = _silu(_ln(h, g[...], bl[...]))
    o_ref[...] = _dot(h, w2[...]) + b2[...]


def _tc_pool_mlp4(pairs, weights, aux, w1a, w1b, b1, pln, p2, blk):
    n, kd = pairs.shape
    k_ = weights.shape[1]
    dout = p2["W"].shape[1]
    if aux is None:
        def body(pairs_ref, w_ref, w1a_, b1_, g_, bl_, w2_, b2_, o_ref):
            _pool_mlp4_body(pairs_ref, w_ref, None, w1a_, None, b1_, g_, bl_,
                            w2_, b2_, o_ref)
        in_specs = [_rows(blk, kd), _rows(blk, k_), _full2((_D, _D)),
                    _full2((1, _D)), _full2((1, _D)), _full2((1, _D)),
                    _full2((_D, dout)), _full2((1, dout))]
        args = (pairs, weights, w1a, b1[None], pln["g"][None], pln["b"][None],
                p2["W"], p2["b"][None])
    else:
        body = _pool_mlp4_body
        da = aux.shape[1]
        in_specs = [_rows(blk, kd), _rows(blk, k_), _rows(blk, da),
                    _full2((_D, _D)), _full2((da, _D)), _full2((1, _D)),
                    _full2((1, _D)), _full2((1, _D)), _full2((_D, dout)),
                    _full2((1, dout))]
        args = (pairs, weights, aux, w1a, w1b, b1[None], pln["g"][None],
                pln["b"][None], p2["W"], p2["b"][None])
    return pl.pallas_call(
        body,
        grid=(n // blk,),
        in_specs=in_specs,
        out_specs=_rows(blk, dout),
        out_shape=jax.ShapeDtypeStruct((n, dout), jnp.float32),
    )(*args)


def _xdxs_body(x_ref, wd, ws, be, xd_ref, xs_ref):
    x = x_ref[...]
    xd_ref[...] = _dot(x, wd[...]) + be[...]
    xs_ref[...] = _dot(x, ws[...])


def _tc_xdxs(x, wd, ws, be, blk):
    n = x.shape[0]
    return pl.pallas_call(
        _xdxs_body,
        grid=(n // blk,),
        in_specs=[_rows(blk, _D), _full2((_D, _D)), _full2((_D, _D)),
                  _full2((1, _D))],
        out_specs=[_rows(blk, _D), _rows(blk, _D)],
        out_shape=[jax.ShapeDtypeStruct((n, _D), jnp.float32),
                   jax.ShapeDtypeStruct((n, _D), jnp.float32)],
    )(x, wd, ws, be[None])


def _msg_body(pre_ref, ea_ref, wea, g, bl, o_ref):
    h = pre_ref[...] + _dot(ea_ref[...], wea[...])
    o_ref[...] = _silu(_ln(h, g[...], bl[...]))


def _tc_msg(pre, ea_p, wea_p, pln, blk):
    n = pre.shape[0]
    da = ea_p.shape[1]
    return pl.pallas_call(
        _msg_body,
        grid=(n // blk,),
        in_specs=[_rows(blk, _D), _rows(blk, da), _full2((da, _D)),
                  _full2((1, _D)), _full2((1, _D))],
        out_specs=_rows(blk, _D),
        out_shape=jax.ShapeDtypeStruct((n, _D), jnp.float32),
    )(pre, ea_p, wea_p, pln["g"][None], pln["b"][None])


def _node_body(x_ref, a0_ref, a1_ref, w1a, w1b, b1, g, bl, w2, b2, o_ref):
    x = x_ref[...]
    agg = a0_ref[...] + a1_ref[...]
    h = _dot(x, w1a[...]) + _dot(agg, w1b[...]) + b1[...]
    h = _silu(_ln(h, g[...], bl[...]))
    o_ref[...] = x + _dot(h, w2[...]) + b2[...]


def _tc_node(x, a0, a1, w1a, w1b, b1, pln, p2, blk):
    n = x.shape[0]
    return pl.pallas_call(
        _node_body,
        grid=(n // blk,),
        in_specs=[_rows(blk, _D)] * 3 + [_full2((_D, _D)), _full2((_D, _D)),
                  _full2((1, _D)), _full2((1, _D)), _full2((1, _D)),
                  _full2((_D, _D)), _full2((1, _D))],
        out_specs=_rows(blk, _D),
        out_shape=jax.ShapeDtypeStruct((n, _D), jnp.float32),
    )(x, a0, a1, w1a, w1b, b1[None], pln["g"][None], pln["b"][None],
      p2["W"], p2["b"][None])


def _segsum_body(dst_ref, msg_ref, o_ref):
    n_i = pl.program_id(0)
    e_i = pl.program_id(1)
    nblk = o_ref.shape[0]
    eblk = msg_ref.shape[0]

    @pl.when(e_i == 0)
    def _():
        o_ref[...] = jnp.zeros_like(o_ref)

    rows = jax.lax.broadcasted_iota(jnp.int32, (nblk, eblk), 0) + n_i * nblk
    onehot_t = (rows == dst_ref[...]).astype(jnp.float32)
    o_ref[...] += _dot(onehot_t, msg_ref[...])


def _tc_segsum(msg, dst, n_seg, nblk, eblk):
    e = msg.shape[0]
    n_seg_p = -(-n_seg // nblk) * nblk
    return pl.pallas_call(
        _segsum_body,
        grid=(n_seg_p // nblk, e // eblk),
        in_specs=[pl.BlockSpec((1, eblk), lambda n, i: (0, i)),
                  pl.BlockSpec((eblk, _D), lambda n, i: (i, 0))],
        out_specs=pl.BlockSpec((nblk, _D), lambda n, i: (n, 0)),
        out_shape=jax.ShapeDtypeStruct((n_seg_p, _D), jnp.float32),
    )(dst[None], msg)[:n_seg]


_SC_MESH = plsc.VectorSubcoreMesh(core_axis_name="c", subcore_axis_name="s")


def _sc_gather(table, idx_flat, ch):
    b = idx_flat.shape[0]
    d = table.shape[1]
    b_per_w = b // _NW
    n_ch = b_per_w // ch

    @functools.partial(
        pl.kernel, mesh=_SC_MESH,
        out_type=jax.ShapeDtypeStruct((b, d), jnp.float32),
        scratch_types=[pltpu.VMEM((ch,), jnp.int32),
                       pltpu.VMEM((ch, d), jnp.float32),
                       pltpu.SemaphoreType.DMA],
    )
    def k(table_hbm, idx_hbm, out_hbm, idx_v, rows_v, sem):
        wid = lax.axis_index("s") * _NC + lax.axis_index("c")
        base = wid * b_per_w

        def body(c, carry):
            off = base + c * ch
            pltpu.sync_copy(idx_hbm.at[pl.ds(off, ch)], idx_v)
            pltpu.async_copy(table_hbm.at[idx_v], rows_v, sem).wait()
            pltpu.sync_copy(rows_v, out_hbm.at[pl.ds(off, ch)])
            return carry

        lax.fori_loop(0, n_ch, body, 0)

    return k(table, idx_flat)


def _sc_gather2_add(xd, xs, dst, src, ch):
    b = dst.shape[0]
    b_per_w = b // _NW
    n_ch = b_per_w // ch

    @functools.partial(
        pl.kernel, mesh=_SC_MESH,
        out_type=jax.ShapeDtypeStruct((b, _D), jnp.float32),
        scratch_types=[pltpu.VMEM((ch,), jnp.int32),
                       pltpu.VMEM((ch, _D), jnp.float32),
                       pltpu.VMEM((ch, _D), jnp.float32),
                       pltpu.SemaphoreType.DMA],
    )
    def k(xd_hbm, xs_hbm, dst_hbm, src_hbm, out_hbm, idx_v, rd, rs, sem):
        wid = lax.axis_index("s") * _NC + lax.axis_index("c")
        base = wid * b_per_w

        def body(c, carry):
            off = base + c * ch
            pltpu.sync_copy(dst_hbm.at[pl.ds(off, ch)], idx_v)
            pltpu.async_copy(xd_hbm.at[idx_v], rd, sem).wait()
            pltpu.sync_copy(src_hbm.at[pl.ds(off, ch)], idx_v)
            pltpu.async_copy(xs_hbm.at[idx_v], rs, sem).wait()

            def add_row(r, carry2):
                for f in range(_D // 16):
                    sl = pl.ds(f * 16, 16)
                    rd[r, sl] = rd[r, sl] + rs[r, sl]
                return carry2

            lax.fori_loop(0, ch, add_row, 0)
            pltpu.sync_copy(rd, out_hbm.at[pl.ds(off, ch)])
            return carry

        lax.fori_loop(0, n_ch, body, 0)

    return k(xd, xs, dst, src)


def _sc_scatter_add(msg, eidx, n_seg_p, ch):
    b = msg.shape[0]
    b_per_w = b // _NW
    n_ch = b_per_w // ch
    rows_per_tile = n_seg_p // _NS
    zcopy = ch
    n_z = rows_per_tile // zcopy
    msg1 = msg.reshape(-1)

    @functools.partial(
        pl.kernel, mesh=_SC_MESH,
        out_type=jax.ShapeDtypeStruct((_NC, n_seg_p * _D), jnp.float32),
        scratch_types=[pltpu.VMEM((ch * _D,), jnp.int32),
                       pltpu.VMEM((ch * _D,), jnp.float32),
                       pltpu.VMEM((zcopy * _D,), jnp.float32),
                       pltpu.VMEM_SHARED((n_seg_p * _D,), jnp.float32),
                       pltpu.SemaphoreType.DMA],
    )
    def k(msg_hbm, eidx_hbm, out_hbm, eidx_v, rows_v, zero_v, acc, sem):
        cid = lax.axis_index("c")
        sid = lax.axis_index("s")
        wid = sid * _NC + cid
        base = wid * b_per_w * _D

        def zf(r, carry):
            zero_v[pl.ds(r * 16, 16)] = jnp.zeros((16,), jnp.float32)
            return carry

        lax.fori_loop(0, zcopy * _D // 16, zf, 0)
        zh = [pltpu.async_copy(
                  zero_v,
                  acc.at[pl.ds((sid * rows_per_tile + j * zcopy) * _D,
                               zcopy * _D)],
                  sem) for j in range(n_z)]
        for h in zh:
            h.wait()
        plsc.subcore_barrier()
        pl.delay(50000)

        def body(c, carry):
            off = base + c * ch * _D
            pltpu.sync_copy(eidx_hbm.at[pl.ds(off, ch * _D)], eidx_v)
            pltpu.sync_copy(msg_hbm.at[pl.ds(off, ch * _D)], rows_v)
            pltpu.sync_copy(rows_v, acc.at[eidx_v], add=True)
            return carry

        lax.fori_loop(0, n_ch, body, 0)
        plsc.subcore_barrier()
        pl.delay(50000)

        dh = [pltpu.async_copy(
                  acc.at[pl.ds((sid * rows_per_tile + j * zcopy) * _D,
                               zcopy * _D)],
                  out_hbm.at[cid].at[pl.ds((sid * rows_per_tile
                                            + j * zcopy) * _D, zcopy * _D)],
                  sem) for j in range(n_z)]
        for h in dh:
            h.wait()

    return k(msg1, eidx)


def _pad_cols(a, to):
    n, c = a.shape
    return jnp.concatenate([a, jnp.zeros((n, to - c), a.dtype)], axis=1)


def kernel(grid_input, mesh_features, edge_index, edge_attr, g2m_indices,
           g2m_weights, m2g_indices, m2g_weights, params):
    bsz, ng, gd = grid_input.shape
    nm = mesh_features.shape[0]
    e = edge_index.shape[1]
    kk = g2m_indices.shape[1]
    p = params

    x_g = grid_input.reshape(ng, gd)

    gp = _tc_mlp4(x_g, p["enc_grid_l1"], p["enc_grid_ln"], p["enc_grid_l2"],
                  blk=2000)

    g2m_flat = g2m_indices.reshape(-1).astype(jnp.int32)
    npad = (-g2m_flat.shape[0]) % (8 * _NW * kk)
    g2m_flat = jnp.concatenate(
        [g2m_flat, jnp.zeros((npad,), jnp.int32)]) if npad else g2m_flat
    pairs = _sc_gather(gp, g2m_flat, ch=256)
    pairs = pairs[:nm * kk].reshape(nm, kk * _D)

    mf_p = _pad_cols(mesh_features, 8)
    w1 = p["enc_comb_l1"]["W"]
    w1a, w1b = w1[:_D], _pad_cols(w1[_D:].T, 8).T
    x = _tc_pool_mlp4(pairs, g2m_weights, mf_p, w1a, w1b,
                      p["enc_comb_l1"]["b"], p["enc_comb_ln"],
                      p["enc_comb_l2"], blk=2000)

    src = edge_index[0].astype(jnp.int32)
    dst = edge_index[1].astype(jnp.int32)
    ea_p = _pad_cols(edge_attr, 8)
    nm_p = -(-nm // (8 * _NS)) * (8 * _NS)
    eidx = (dst[:, None] * _D + jnp.arange(_D, dtype=jnp.int32)[None]).reshape(-1)

    for lp in p["layers"]:
        we = lp["edge_l"]["W"]
        wd, ws = we[:_D], we[_D:2 * _D]
        wea_p = _pad_cols(we[2 * _D:].T, 8).T
        xd, xs = _tc_xdxs(x, wd, ws, lp["edge_l"]["b"], blk=2000)
        pre = _sc_gather2_add(xd, xs, dst, src, ch=200)
        msg = _tc_msg(pre, ea_p, wea_p, lp["edge_ln"], blk=2000)
        agg = _tc_segsum(msg, dst, nm, nblk=2048, eblk=3200)
        wn = lp["node_l1"]["W"]
        x = _tc_node(x, agg, jnp.zeros_like(agg), wn[:_D], wn[_D:],
                     lp["node_l1"]["b"], lp["node_ln"], lp["node_l2"],
                     blk=2000)

    m2g_flat = m2g_indices.reshape(-1).astype(jnp.int32)
    pairs2 = _sc_gather(x, m2g_flat, ch=200)
    pairs2 = pairs2.reshape(ng, kk * _D)
    out = _tc_pool_mlp4(pairs2, m2g_weights, None, p["dec_l1"]["W"], None,
                        p["dec_l1"]["b"], p["dec_ln"], p["dec_l2"], blk=2000)
    return out.reshape(bsz, ng, gd)
